# Initial kernel scaffold; baseline (speedup 1.0000x reference)
import jax
import jax.numpy as jnp
from jax.experimental import pallas as pl


def kernel(states, W_embed_init, W_embed_current, W_pos):
    raise NotImplementedError("write your pallas kernel here")


# speedup vs baseline: 3.3936x; 3.3936x over previous
import functools

import jax
import jax.numpy as jnp
from jax import lax
from jax.experimental import pallas as pl
from jax.experimental.pallas import tpu as pltpu
from jax.experimental.pallas import tpu_sc as plsc

B = 512
P = 512
D = 256
V = 21
L = 16

NC = 2
NS = 16
NW = NC * NS
PW = P // NW
CB = 8
ROWS = CB * PW
NCHUNK = B // CB


def _sc_embed(idx_arr, table, wpos):
    mesh = plsc.VectorSubcoreMesh(core_axis_name="c", subcore_axis_name="s")

    @functools.partial(
        pl.kernel,
        mesh=mesh,
        out_type=jax.ShapeDtypeStruct((B, P, D), jnp.float32),
        scratch_types=[
            pltpu.VMEM((NCHUNK, ROWS), jnp.int32),
            pltpu.VMEM((PW, D), jnp.float32),
            pltpu.VMEM((ROWS, D), jnp.float32),
            pltpu.SemaphoreType.DMA,
        ],
    )
    def k(idx_hbm, table_hbm, wpos_hbm, out_hbm, idx_v, pos_v, buf, sem):
        wid = lax.axis_index("s") * NC + lax.axis_index("c")
        pltpu.sync_copy(idx_hbm.at[wid], idx_v)
        pltpu.sync_copy(wpos_hbm.at[pl.ds(wid * PW, PW)], pos_v)

        def chunk(g, carry):
            pltpu.async_copy(table_hbm.at[idx_v.at[g]], buf, sem).wait()

            def add_pos(j, c2):
                for c in range(D // L):
                    pv = pos_v[j, pl.ds(c * L, L)]
                    for bb in range(CB):
                        r = bb * PW + j
                        buf[r, pl.ds(c * L, L)] = buf[r, pl.ds(c * L, L)] + pv
                return c2

            lax.fori_loop(0, PW, add_pos, 0)
            for bb in range(CB):
                pltpu.sync_copy(
                    buf.at[pl.ds(bb * PW, PW), :],
                    out_hbm.at[g * CB + bb, pl.ds(wid * PW, PW), :],
                )
            return carry

        lax.fori_loop(0, NCHUNK, chunk, 0)

    return k(idx_arr, table, wpos)


def kernel(states, W_embed_init, W_embed_current, W_pos):
    cidx = states[:, :P].astype(jnp.int32) * V + states[:, P:].astype(jnp.int32)
    carr = (
        cidx.T.reshape(NW, PW, B).transpose(0, 2, 1).reshape(NW, NCHUNK, ROWS)
    )
    table = (W_embed_init[:, None, :] + W_embed_current[None, :, :]).reshape(
        V * V, D
    )
    return _sc_embed(carr, table, W_pos)


# speedup vs baseline: 5.5097x; 1.6235x over previous
import functools

import jax
import jax.numpy as jnp
from jax import lax
from jax.experimental import pallas as pl
from jax.experimental.pallas import tpu as pltpu
from jax.experimental.pallas import tpu_sc as plsc

B = 512
P = 512
D = 256
V = 21
L = 16

NC = 2
NS = 16
NW = NC * NS
PW = P // NW
CB = 4
ROWS = CB * PW
NCHUNK = B // CB
NBUF = 4


def _sc_embed(idx_arr, table, wpos):
    mesh = plsc.VectorSubcoreMesh(core_axis_name="c", subcore_axis_name="s")

    @functools.partial(
        pl.kernel,
        mesh=mesh,
        out_type=jax.ShapeDtypeStruct((B, P, D), jnp.float32),
        scratch_types=[
            pltpu.VMEM((NCHUNK, ROWS), jnp.int32),
            pltpu.VMEM((PW, D), jnp.float32),
            pltpu.VMEM((ROWS, D), jnp.float32),
            pltpu.VMEM((ROWS, D), jnp.float32),
            pltpu.VMEM((ROWS, D), jnp.float32),
            pltpu.VMEM((ROWS, D), jnp.float32),
            pltpu.SemaphoreType.DMA((NBUF,)),
            pltpu.SemaphoreType.DMA((NBUF,)),
        ],
    )
    def k(idx_hbm, table_hbm, wpos_hbm, out_hbm, idx_v, pos_v, r0, r1, r2, r3,
          gsem, wsem):
        bufs = (r0, r1, r2, r3)
        wid = lax.axis_index("s") * NC + lax.axis_index("c")
        pltpu.sync_copy(idx_hbm.at[wid], idx_v)
        pltpu.sync_copy(wpos_hbm.at[pl.ds(wid * PW, PW)], pos_v)

        def gather(g, b):
            return pltpu.make_async_copy(
                table_hbm.at[idx_v.at[g]], bufs[b], gsem.at[b]
            )

        def writes(g, b):
            return [
                pltpu.make_async_copy(
                    bufs[b].at[pl.ds(bb * PW, PW), :],
                    out_hbm.at[g * CB + bb, pl.ds(wid * PW, PW), :],
                    wsem.at[b],
                )
                for bb in range(CB)
            ]

        for b in range(NBUF - 1):
            gather(b, b).start()

        def body(t, carry):
            for b in range(NBUF):
                g = t * NBUF + b
                gather(g, b).wait()

                def add_pos(j, c2, _b=b):
                    for c in range(D // L):
                        pv = pos_v[j, pl.ds(c * L, L)]
                        for bb in range(CB):
                            r = bb * PW + j
                            sl = pl.ds(c * L, L)
                            bufs[_b][r, sl] = bufs[_b][r, sl] + pv
                    return c2

                lax.fori_loop(0, PW, add_pos, 0)
                for wcp in writes(g, b):
                    wcp.start()
                nb = (b + NBUF - 1) % NBUF
                if b == 0:
                    @pl.when(t > 0)
                    def _():
                        for wcp in writes(g - 1, nb):
                            wcp.wait()
                    gather(g + NBUF - 1, nb).start()
                else:
                    for wcp in writes(g - 1, nb):
                        wcp.wait()

                    @pl.when(t < NCHUNK // NBUF - 1)
                    def _():
                        gather(g + NBUF - 1, nb).start()
            return carry

        lax.fori_loop(0, NCHUNK // NBUF, body, 0)
        for wcp in writes(NCHUNK - 1, NBUF - 1):
            wcp.wait()

    return k(idx_arr, table, wpos)


def kernel(states, W_embed_init, W_embed_current, W_pos):
    cidx = states[:, :P].astype(jnp.int32) * V + states[:, P:].astype(jnp.int32)
    carr = (
        cidx.T.reshape(NW, PW, B).transpose(0, 2, 1).reshape(NW, NCHUNK, ROWS)
    )
    table = (W_embed_init[:, None, :] + W_embed_current[None, :, :]).reshape(
        V * V, D
    )
    return _sc_embed(carr, table, W_pos)
